# baseline (device time: 109120 ns/iter reference)
import jax
import jax.numpy as jnp
from jax import lax
from jax.experimental import pallas as pl
from jax.experimental.pallas import tpu as pltpu

N_CHUNK = 8
SEND_SLOTS = 4
RLAG = 2
PAD = 32
SIZES = [160, 544, 576, 576, 576, 576, 576, 512]


def kernel(x, pi):
    _, m, n = x.shape
    assert sum(SIZES) == m and len(SIZES) == N_CHUNK
    offs = [sum(SIZES[:c]) for c in range(N_CHUNK)]
    roffs = [offs[c] + c * PAD for c in range(N_CHUNK)]
    rows = max(SIZES)

    def body(
        pi_ref,
        x_hbm,
        out_hbm,
        in_stage,
        send_q,
        recv_q,
        out_stage,
        in_sems,
        send_sems,
        recv_sems,
        out_sems,
    ):
        my_x = lax.axis_index("x")
        my_y = lax.axis_index("y")
        dst_y = pi_ref[my_y]

        x_at = lambda c: x_hbm.at[0, pl.ds(offs[c], SIZES[c]), :]
        out_at = lambda c: out_hbm.at[0, pl.ds(offs[c], SIZES[c]), :]
        in_at = lambda c: in_stage.at[c % 2, pl.ds(0, SIZES[c])]
        outst_at = lambda c: out_stage.at[c % 2, pl.ds(0, SIZES[c])]
        fetches = [
            pltpu.make_async_copy(x_at(c), in_at(c), in_sems.at[c % 2])
            for c in range(N_CHUNK)
        ]

        @pl.when(dst_y == my_y)
        def _():
            for c in range(N_CHUNK):
                fetches[c].start()
                fetches[c].wait()
                outst_at(c)[...] = in_at(c)[...].astype(jnp.bfloat16)
                st = pltpu.make_async_copy(
                    outst_at(c), out_at(c), out_sems.at[c % 2]
                )
                st.start()
                st.wait()

        @pl.when(dst_y != my_y)
        def _():
            def data_rdma(c):
                return pltpu.make_async_remote_copy(
                    src_ref=send_q.at[c % SEND_SLOTS, pl.ds(0, PAD + SIZES[c])],
                    dst_ref=recv_q.at[pl.ds(roffs[c], PAD + SIZES[c])],
                    send_sem=send_sems.at[c % SEND_SLOTS],
                    recv_sem=recv_sems.at[c],
                    device_id=(my_x, dst_y),
                    device_id_type=pl.DeviceIdType.MESH,
                )

            drdmas = [data_rdma(c) for c in range(N_CHUNK)]
            stores = []

            def process_recv(j):
                drdmas[j].wait_recv()
                if j >= 2:
                    stores[j - 2].wait()
                hdr = recv_q[pl.ds(roffs[j], PAD), pl.ds(0, 128)]
                s = pltpu.bitcast(hdr, jnp.float32)[0, 0]
                outst_at(j)[...] = (
                    recv_q[pl.ds(roffs[j] + PAD, SIZES[j]), :].astype(jnp.float32)
                    * s
                ).astype(jnp.bfloat16)
                st = pltpu.make_async_copy(
                    outst_at(j), out_at(j), out_sems.at[j % 2]
                )
                st.start()
                stores.append(st)

            fetches[0].start()
            barrier_sem = pltpu.get_barrier_semaphore()
            pl.semaphore_signal(
                barrier_sem, inc=1,
                device_id=(my_x, dst_y), device_id_type=pl.DeviceIdType.MESH,
            )

            for c in range(N_CHUNK):
                fetches[c].wait()
                if c + 1 < N_CHUNK:
                    fetches[c + 1].start()
                if c >= SEND_SLOTS:
                    drdmas[c - SEND_SLOTS].wait_send()
                chunk = in_at(c)[...]
                absmax = jnp.max(jnp.abs(chunk))
                inv = 127.0 / jnp.maximum(absmax, 1e-30)
                send_q[c % SEND_SLOTS, pl.ds(PAD, SIZES[c]), :] = jnp.clip(
                    jnp.round(chunk * inv), -127.0, 127.0
                ).astype(jnp.int8)
                send_q[c % SEND_SLOTS, pl.ds(0, PAD), pl.ds(0, 128)] = (
                    pltpu.bitcast(
                        jnp.full((PAD // 4, 128), absmax * (1.0 / 127.0), jnp.float32),
                        jnp.int8,
                    )
                )
                if c == 0:
                    pl.semaphore_wait(barrier_sem, 1)
                drdmas[c].start()
                if c >= RLAG:
                    process_recv(c - RLAG)

            for j in range(N_CHUNK - RLAG, N_CHUNK):
                process_recv(j)
            stores[N_CHUNK - 2].wait()
            stores[N_CHUNK - 1].wait()
            for c in range(max(0, N_CHUNK - SEND_SLOTS), N_CHUNK):
                drdmas[c].wait_send()

    return pl.pallas_call(
        body,
        out_shape=jax.ShapeDtypeStruct(x.shape, jnp.bfloat16),
        in_specs=[
            pl.BlockSpec(memory_space=pltpu.SMEM),
            pl.BlockSpec(memory_space=pl.ANY),
        ],
        out_specs=pl.BlockSpec(memory_space=pl.ANY),
        scratch_shapes=[
            pltpu.VMEM((2, rows, n), jnp.float32),
            pltpu.VMEM((SEND_SLOTS, PAD + rows, n), jnp.int8),
            pltpu.VMEM((m + N_CHUNK * PAD, n), jnp.int8),
            pltpu.VMEM((2, rows, n), jnp.bfloat16),
            pltpu.SemaphoreType.DMA((2,)),
            pltpu.SemaphoreType.DMA((SEND_SLOTS,)),
            pltpu.SemaphoreType.DMA((N_CHUNK,)),
            pltpu.SemaphoreType.DMA((2,)),
        ],
        compiler_params=pltpu.CompilerParams(collective_id=0),
    )(pi, x)


# device time: 103950 ns/iter; 1.0497x vs baseline; 1.0497x over previous
import jax
import jax.numpy as jnp
from jax import lax
from jax.experimental import pallas as pl
from jax.experimental.pallas import tpu as pltpu

N_CHUNK = 8
SEND_SLOTS = 4
RLAG = 2
SIZES = [160, 544, 576, 576, 576, 576, 576, 512]


def kernel(x, pi):
    _, m, n = x.shape
    assert sum(SIZES) == m and len(SIZES) == N_CHUNK
    offs = [sum(SIZES[:c]) for c in range(N_CHUNK)]
    rows = max(SIZES)

    def body(
        pi_ref,
        x_hbm,
        out_hbm,
        in_stage,
        send_q,
        scale_send,
        recv_q,
        scale_recv,
        out_stage,
        in_sems,
        send_sems,
        ssend_sems,
        recv_sems,
        srecv_sems,
        out_sems,
    ):
        my_x = lax.axis_index("x")
        my_y = lax.axis_index("y")
        dst_y = pi_ref[my_y]

        x_at = lambda c: x_hbm.at[0, pl.ds(offs[c], SIZES[c]), :]
        out_at = lambda c: out_hbm.at[0, pl.ds(offs[c], SIZES[c]), :]
        recv_at = lambda c: recv_q.at[pl.ds(offs[c], SIZES[c]), :]
        in_at = lambda c: in_stage.at[c % 2, pl.ds(0, SIZES[c])]
        sendq_at = lambda c: send_q.at[c % SEND_SLOTS, pl.ds(0, SIZES[c])]
        outst_at = lambda c: out_stage.at[c % 2, pl.ds(0, SIZES[c])]
        fetches = [
            pltpu.make_async_copy(x_at(c), in_at(c), in_sems.at[c % 2])
            for c in range(N_CHUNK)
        ]

        @pl.when(dst_y == my_y)
        def _():
            for c in range(N_CHUNK):
                fetches[c].start()
                fetches[c].wait()
                outst_at(c)[...] = in_at(c)[...].astype(jnp.bfloat16)
                st = pltpu.make_async_copy(
                    outst_at(c), out_at(c), out_sems.at[c % 2]
                )
                st.start()
                st.wait()

        @pl.when(dst_y != my_y)
        def _():
            def data_rdma(c):
                return pltpu.make_async_remote_copy(
                    src_ref=sendq_at(c),
                    dst_ref=recv_at(c),
                    send_sem=send_sems.at[c % SEND_SLOTS],
                    recv_sem=recv_sems.at[c],
                    device_id=(my_x, dst_y),
                    device_id_type=pl.DeviceIdType.MESH,
                )

            def scale_rdma(c):
                return pltpu.make_async_remote_copy(
                    src_ref=scale_send.at[c % SEND_SLOTS],
                    dst_ref=scale_recv.at[c],
                    send_sem=ssend_sems.at[c % SEND_SLOTS],
                    recv_sem=srecv_sems.at[c],
                    device_id=(my_x, dst_y),
                    device_id_type=pl.DeviceIdType.MESH,
                )

            drdmas = [data_rdma(c) for c in range(N_CHUNK)]
            srdmas = [scale_rdma(c) for c in range(N_CHUNK)]
            stores = []

            def process_recv(j):
                drdmas[j].wait_recv()
                srdmas[j].wait_recv()
                if j >= 2:
                    stores[j - 2].wait()
                s = scale_recv[j, 0, 0]
                outst_at(j)[...] = (
                    recv_at(j)[...].astype(jnp.float32) * s
                ).astype(jnp.bfloat16)
                st = pltpu.make_async_copy(
                    outst_at(j), out_at(j), out_sems.at[j % 2]
                )
                st.start()
                stores.append(st)

            fetches[0].start()
            barrier_sem = pltpu.get_barrier_semaphore()
            pl.semaphore_signal(
                barrier_sem, inc=1,
                device_id=(my_x, dst_y), device_id_type=pl.DeviceIdType.MESH,
            )

            for c in range(N_CHUNK):
                fetches[c].wait()
                if c + 1 < N_CHUNK:
                    fetches[c + 1].start()
                if c >= SEND_SLOTS:
                    drdmas[c - SEND_SLOTS].wait_send()
                    srdmas[c - SEND_SLOTS].wait_send()
                chunk = in_at(c)[...]
                absmax = jnp.max(jnp.abs(chunk))
                inv = 127.0 / jnp.maximum(absmax, 1e-30)
                sendq_at(c)[...] = jnp.clip(
                    jnp.round(chunk * inv), -127.0, 127.0
                ).astype(jnp.int8)
                scale_send[c % SEND_SLOTS] = jnp.full(
                    (8, 128), absmax * (1.0 / 127.0), jnp.float32
                )
                if c == 0:
                    pl.semaphore_wait(barrier_sem, 1)
                drdmas[c].start()
                srdmas[c].start()
                if c >= RLAG:
                    process_recv(c - RLAG)

            for j in range(N_CHUNK - RLAG, N_CHUNK):
                process_recv(j)
            stores[N_CHUNK - 2].wait()
            stores[N_CHUNK - 1].wait()
            for c in range(max(0, N_CHUNK - SEND_SLOTS), N_CHUNK):
                drdmas[c].wait_send()
                srdmas[c].wait_send()

    return pl.pallas_call(
        body,
        out_shape=jax.ShapeDtypeStruct(x.shape, jnp.bfloat16),
        in_specs=[
            pl.BlockSpec(memory_space=pltpu.SMEM),
            pl.BlockSpec(memory_space=pl.ANY),
        ],
        out_specs=pl.BlockSpec(memory_space=pl.ANY),
        scratch_shapes=[
            pltpu.VMEM((2, rows, n), jnp.float32),
            pltpu.VMEM((SEND_SLOTS, rows, n), jnp.int8),
            pltpu.VMEM((SEND_SLOTS, 8, 128), jnp.float32),
            pltpu.VMEM((m, n), jnp.int8),
            pltpu.VMEM((N_CHUNK, 8, 128), jnp.float32),
            pltpu.VMEM((2, rows, n), jnp.bfloat16),
            pltpu.SemaphoreType.DMA((2,)),
            pltpu.SemaphoreType.DMA((SEND_SLOTS,)),
            pltpu.SemaphoreType.DMA((SEND_SLOTS,)),
            pltpu.SemaphoreType.DMA((N_CHUNK,)),
            pltpu.SemaphoreType.DMA((N_CHUNK,)),
            pltpu.SemaphoreType.DMA((2,)),
        ],
        compiler_params=pltpu.CompilerParams(collective_id=0),
    )(pi, x)
